# baseline (speedup 1.0000x reference)
import functools

import jax
import jax.numpy as jnp
from jax import lax
from jax.experimental import pallas as pl
from jax.experimental.pallas import tpu as pltpu
from jax.experimental.pallas import tpu_sc as plsc

N = 10000
E = 320000
D = 128
H = 128
G = 16

NC = 2
NS = 16
NW = NC * NS
EPW = E // NW
EPT = E // NS
SLICES = ((0, 38400), (38400, 153600), (192000, 128000))
CHG = 200
CHS = 160
CHT = 160
NP = 10240
RPT = NP // NS

@functools.cache
def _sc_mesh():
    return plsc.VectorSubcoreMesh(
        core_axis_name="c", subcore_axis_name="s", num_cores=NC, num_subcores=NS)


def _prep_body(x_ref, ws_ref, wr_ref, be1_ref, p_ref, q_ref):
    xv = x_ref[...]
    p_ref[...] = jnp.dot(xv, ws_ref[...], preferred_element_type=jnp.float32)
    q_ref[...] = (jnp.dot(xv, wr_ref[...], preferred_element_type=jnp.float32)
                  + be1_ref[...])


def _prep(x, w1s, w1r, be1):
    bn = 2000
    return pl.pallas_call(
        _prep_body,
        grid=(N // bn,),
        in_specs=[
            pl.BlockSpec((bn, D), lambda i: (i, 0)),
            pl.BlockSpec((D, H), lambda i: (0, 0)),
            pl.BlockSpec((D, H), lambda i: (0, 0)),
            pl.BlockSpec((1, H), lambda i: (0, 0)),
        ],
        out_specs=[
            pl.BlockSpec((bn, H), lambda i: (i, 0)),
            pl.BlockSpec((bn, H), lambda i: (i, 0)),
        ],
        out_shape=[
            jax.ShapeDtypeStruct((N, H), jnp.float32),
            jax.ShapeDtypeStruct((N, H), jnp.float32),
        ],
    )(x, w1s, w1r, be1.reshape(1, H))


@functools.cache
def _sc_gather_kernel(ebase, esize):
    EPW_S = esize // NW
    NCHG = EPW_S // CHG
    assert NCHG * CHG == EPW_S and NCHG % 2 == 0

    @functools.partial(
        pl.kernel,
        out_type=jax.ShapeDtypeStruct((esize, H), jnp.float32),
        mesh=_sc_mesh(),
        scratch_types=[
            pltpu.VMEM((CHG,), jnp.int32),
            pltpu.VMEM((CHG,), jnp.int32),
            pltpu.VMEM((CHG,), jnp.int32),
            pltpu.VMEM((CHG,), jnp.int32),
            pltpu.VMEM((CHG, H), jnp.float32),
            pltpu.VMEM((CHG, H), jnp.float32),
            pltpu.VMEM((CHG, H), jnp.float32),
            pltpu.VMEM((CHG, H), jnp.float32),
            pltpu.SemaphoreType.DMA,
            pltpu.SemaphoreType.DMA,
            pltpu.SemaphoreType.DMA,
            pltpu.SemaphoreType.DMA,
        ],
    )
    def _sc_gather_impl(p_hbm, q_hbm, snd_hbm, rcv_hbm, s_hbm,
                        is0, is1, ir0, ir1, a0, a1, b0, b1,
                        gs0, gs1, ws0, ws1):
        IS, IR, A, B = (is0, is1), (ir0, ir1), (a0, a1), (b0, b1)
        GS, WS = (gs0, gs1), (ws0, ws1)
        wid = lax.axis_index("s") * NC + lax.axis_index("c")
        base = wid * EPW_S

        def load_idx(i, b):
            off = ebase + base + i * CHG
            pltpu.sync_copy(snd_hbm.at[pl.ds(off, CHG)], IS[b])
            pltpu.sync_copy(rcv_hbm.at[pl.ds(off, CHG)], IR[b])

        def start_gather(b):
            pltpu.async_copy(p_hbm.at[IS[b]], A[b], GS[b])
            pltpu.async_copy(q_hbm.at[IR[b]], B[b], GS[b])

        def wait_gather(b):
            pltpu.make_async_copy(p_hbm.at[IS[b]], A[b], GS[b]).wait()
            pltpu.make_async_copy(q_hbm.at[IR[b]], B[b], GS[b]).wait()

        def start_wb(i, b):
            off = base + i * CHG
            pltpu.async_copy(A[b], s_hbm.at[pl.ds(off, CHG)], WS[b])

        def wait_wb(b):
            pltpu.make_async_copy(A[b], s_hbm.at[pl.ds(base, CHG)], WS[b]).wait()

        def add(b):
            @plsc.parallel_loop(0, CHG, unroll=4)
            def _row(r):
                for j in range(H // 16):
                    sl = pl.ds(j * 16, 16)
                    A[b][r, sl] = A[b][r, sl] + B[b][r, sl]

        load_idx(0, 0)
        start_gather(0)
        load_idx(1, 1)
        start_gather(1)

        def pair(jj, carry):
            for b in (0, 1):
                wait_gather(b)
                add(b)
                start_wb(2 * jj + b, b)
            for b in (0, 1):
                nxt = 2 * jj + 2 + b

                @pl.when(nxt < NCHG)
                def _():
                    load_idx(nxt, b)
                    wait_wb(b)
                    start_gather(b)
            return carry

        lax.fori_loop(0, NCHG // 2, pair, 0)
        wait_wb(0)
        wait_wb(1)

    return _sc_gather_impl


def _sc_gather(p, q, senders, receivers, ebase, esize):
    return _sc_gather_kernel(ebase, esize)(p, q, senders, receivers)


def _edge_body(ea_ref, s_ref, w1_ref, w2_ref, b2_ref, w3_ref, b3_ref,
               msg_ref, eao_ref):
    ea = ea_ref[...]
    h = (jnp.dot(ea, w1_ref[...], preferred_element_type=jnp.float32)
         + s_ref[...])
    h = jnp.maximum(h, 0.0)
    h = jnp.dot(h, w2_ref[...], preferred_element_type=jnp.float32) + b2_ref[...]
    h = jnp.maximum(h, 0.0)
    m = jnp.dot(h, w3_ref[...], preferred_element_type=jnp.float32) + b3_ref[...]
    msg_ref[...] = m
    eao_ref[...] = ea + m


def _edge(edge_attr, s, w1e, We2, be2, We3, be3, ebase, esize):
    be = 2560
    bb = ebase // be
    return pl.pallas_call(
        _edge_body,
        grid=(esize // be,),
        in_specs=[
            pl.BlockSpec((be, D), lambda i: (i + bb, 0)),
            pl.BlockSpec((be, H), lambda i: (i, 0)),
            pl.BlockSpec((D, H), lambda i: (0, 0)),
            pl.BlockSpec((H, H), lambda i: (0, 0)),
            pl.BlockSpec((1, H), lambda i: (0, 0)),
            pl.BlockSpec((H, D), lambda i: (0, 0)),
            pl.BlockSpec((1, D), lambda i: (0, 0)),
        ],
        out_specs=[
            pl.BlockSpec((be, D), lambda i: (i, 0)),
            pl.BlockSpec((be, D), lambda i: (i, 0)),
        ],
        out_shape=[
            jax.ShapeDtypeStruct((esize, D), jnp.float32),
            jax.ShapeDtypeStruct((esize, D), jnp.float32),
        ],
    )(edge_attr, s, w1e, We2, be2.reshape(1, H), We3, be3.reshape(1, D))


@functools.cache
def _sc_scatter_kernel(ebase, esize):
    EPT_S = esize // NS
    NCHS = (EPT_S // CHS) // 2 * 2
    CHT_S = EPT_S - NCHS * CHS
    assert CHT_S in (0, 80, 160)

    @functools.partial(
        pl.kernel,
        out_type=jax.ShapeDtypeStruct((NC, NP, D), jnp.float32),
        mesh=_sc_mesh(),
        scratch_types=[
            pltpu.VMEM_SHARED((NP, D), jnp.float32),
            pltpu.VMEM((CHS,), jnp.int32),
            pltpu.VMEM((CHS,), jnp.int32),
            pltpu.VMEM((CHT,), jnp.int32),
            pltpu.VMEM((CHS, D), jnp.float32),
            pltpu.VMEM((CHS, D), jnp.float32),
            pltpu.SemaphoreType.DMA,
            pltpu.SemaphoreType.DMA,
            pltpu.SemaphoreType.DMA,
            pltpu.SemaphoreType.DMA,
        ],
    )
    def _sc_scatter_impl(msg_hbm, idx2_hbm, init_hbm, out_hbm,
                         acc, ix0, ix1, ixt, m0, m1, ls0, ls1, ss0, ss1):
        IX, M = (ix0, ix1), (m0, m1)
        LS, SS = (ls0, ls1), (ss0, ss1)
        cid = lax.axis_index("c")
        sid = lax.axis_index("s")

        pltpu.sync_copy(init_hbm.at[cid, pl.ds(sid * RPT, RPT)],
                        acc.at[pl.ds(sid * RPT, RPT)])
        plsc.subcore_barrier()

        base = sid * EPT_S

        def start_load(i, b):
            off = base + i * CHS
            pltpu.async_copy(idx2_hbm.at[pl.ds(cid * E + ebase + off, CHS)],
                             IX[b], LS[b])
            pltpu.async_copy(msg_hbm.at[pl.ds(off, CHS)], M[b], LS[b])

        def wait_load(b):
            pltpu.make_async_copy(idx2_hbm.at[pl.ds(base, CHS)], IX[b], LS[b]).wait()
            pltpu.make_async_copy(msg_hbm.at[pl.ds(base, CHS)], M[b], LS[b]).wait()

        def start_scatter(b):
            pltpu.async_copy(M[b], acc.at[IX[b]], SS[b], add=True)

        def wait_scatter(b):
            pltpu.make_async_copy(M[b], acc.at[IX[b]], SS[b]).wait()

        start_load(0, 0)
        start_load(1, 1)

        def pair(jj, carry):
            for b in (0, 1):
                wait_load(b)
                start_scatter(b)
            for b in (0, 1):
                nxt = 2 * jj + 2 + b

                @pl.when(nxt < NCHS)
                def _():
                    wait_scatter(b)
                    start_load(nxt, b)
            return carry

        lax.fori_loop(0, NCHS // 2, pair, 0)
        wait_scatter(0)
        wait_scatter(1)
        if CHT_S:
            toff = base + NCHS * CHS
            pltpu.sync_copy(idx2_hbm.at[pl.ds(cid * E + ebase + toff, CHT_S)],
                            ixt.at[pl.ds(0, CHT_S)])
            pltpu.sync_copy(msg_hbm.at[pl.ds(toff, CHT_S)],
                            m0.at[pl.ds(0, CHT_S)])
            pltpu.sync_copy(m0.at[pl.ds(0, CHT_S)], acc.at[ixt.at[pl.ds(0, CHT_S)]],
                            add=True)
        plsc.subcore_barrier()

        pltpu.sync_copy(acc.at[pl.ds(sid * RPT, RPT)],
                        out_hbm.at[cid, pl.ds(sid * RPT, RPT)])

    return _sc_scatter_impl


def _sc_scatter(messages, idx2, init, ebase, esize):
    return _sc_scatter_kernel(ebase, esize)(messages, idx2, init)


def _global_body(x_ref, b_ref, u_ref, g1x_ref, g1u_ref, bg1_ref,
                 g2_ref, bg2_ref, g3_ref, bg3_ref, mu_ref):
    xv = x_ref[...]
    b = b_ref[...]
    oh = (b == lax.broadcasted_iota(jnp.int32, (b.shape[0], D), 1)
          ).astype(jnp.float32)
    ub = jnp.dot(oh, u_ref[...], preferred_element_type=jnp.float32)
    h = (jnp.dot(xv, g1x_ref[...], preferred_element_type=jnp.float32)
         + jnp.dot(ub, g1u_ref[...], preferred_element_type=jnp.float32)
         + bg1_ref[...])
    h = jnp.maximum(h, 0.0)
    h = jnp.dot(h, g2_ref[...], preferred_element_type=jnp.float32) + bg2_ref[...]
    h = jnp.maximum(h, 0.0)
    mu_ref[...] = (jnp.dot(h, g3_ref[...], preferred_element_type=jnp.float32)
                   + bg3_ref[...])


def _global(x, batch2d, u_pad, g1x, g1u, bg1, Wg2, bg2, Wg3, bg3):
    bn = 2000
    full = lambda shape: pl.BlockSpec(shape, lambda i: tuple(0 for _ in shape))
    return pl.pallas_call(
        _global_body,
        grid=(N // bn,),
        in_specs=[
            pl.BlockSpec((bn, D), lambda i: (i, 0)),
            pl.BlockSpec((bn, 1), lambda i: (i, 0)),
            full((D, D)),
            full((D, H)), full((D, H)), full((1, H)),
            full((H, H)), full((1, H)),
            full((H, D)), full((1, D)),
        ],
        out_specs=pl.BlockSpec((bn, H), lambda i: (i, 0)),
        out_shape=jax.ShapeDtypeStruct((N, H), jnp.float32),
    )(x, batch2d, u_pad,
      g1x, g1u, bg1.reshape(1, H), Wg2, bg2.reshape(1, H),
      Wg3, bg3.reshape(1, D))


def _node_body(x_ref, rcv_ref, b_ref, mu_ref,
               n1x_ref, n1r_ref, n1u_ref, bn1_ref, n2_ref, bn2_ref,
               n3_ref, bn3_ref, out_ref):
    del b_ref
    xv = x_ref[...]
    rcv = rcv_ref[0] - rcv_ref[1]
    mu = mu_ref[...]
    h = (jnp.dot(xv, n1x_ref[...], preferred_element_type=jnp.float32)
         + jnp.dot(rcv, n1r_ref[...], preferred_element_type=jnp.float32)
         + jnp.dot(mu, n1u_ref[...], preferred_element_type=jnp.float32)
         + bn1_ref[...])
    h = jnp.maximum(h, 0.0)
    h = jnp.dot(h, n2_ref[...], preferred_element_type=jnp.float32) + bn2_ref[...]
    h = jnp.maximum(h, 0.0)
    gx = jnp.dot(h, n3_ref[...], preferred_element_type=jnp.float32) + bn3_ref[...]
    out_ref[...] = gx + xv


def _node(x, rcv, batch2d, mu,
          n1x, n1r, n1u, bn1, Wn2, bn2, Wn3, bn3):
    bn = 2000
    full = lambda shape: pl.BlockSpec(shape, lambda i: tuple(0 for _ in shape))
    return pl.pallas_call(
        _node_body,
        grid=(N // bn,),
        in_specs=[
            pl.BlockSpec((bn, D), lambda i: (i, 0)),
            pl.BlockSpec((NC, bn, D), lambda i: (0, i, 0)),
            pl.BlockSpec((bn, 1), lambda i: (i, 0)),
            pl.BlockSpec((bn, H), lambda i: (i, 0)),
            full((D, H)), full((D, H)), full((D, H)), full((1, H)),
            full((H, H)), full((1, H)),
            full((H, D)), full((1, D)),
        ],
        out_specs=pl.BlockSpec((bn, D), lambda i: (i, 0)),
        out_shape=jax.ShapeDtypeStruct((N, D), jnp.float32),
    )(x, rcv, batch2d, mu,
      n1x, n1r, n1u, bn1.reshape(1, H), Wn2, bn2.reshape(1, H),
      Wn3, bn3.reshape(1, D))


def kernel(x, senders, receivers, edge_attr, u, batch,
           We1, be1, We2, be2, We3, be3,
           Wg1, bg1, Wg2, bg2, Wg3, bg3,
           Wn1, bn1, Wn2, bn2, Wn3, bn3):
    w1e, w1s, w1r = We1[:D], We1[D:2 * D], We1[2 * D:]
    g1x, g1u = Wg1[:D], Wg1[D:]
    n1x, n1r, n1u = Wn1[:D], Wn1[D:2 * D], Wn1[2 * D:]

    p, q = _prep(x, w1s, w1r, be1)
    idx2 = jnp.concatenate([receivers, senders])

    ss = [_sc_gather(p, q, senders, receivers, b, z) for b, z in SLICES]
    eaos = []
    part = jnp.zeros((NC, NP, D), jnp.float32)
    for (b, z), s in zip(SLICES, ss):
        m, eao = _edge(edge_attr, s, w1e, We2, be2, We3, be3, b, z)
        part = _sc_scatter(m, idx2, part, b, z)
        eaos.append(eao)
    edge_attr_update = jnp.concatenate(eaos, axis=0)

    u_pad = jnp.zeros((D, D), jnp.float32).at[:G].set(u)
    batch2d = batch.reshape(N, 1)
    mu = _global(x, batch2d, u_pad, g1x, g1u, bg1, Wg2, bg2, Wg3, bg3)
    x_update = _node(x, part, batch2d, mu,
                     n1x, n1r, n1u, bn1, Wn2, bn2, Wn3, bn3)
    return (x_update, edge_attr_update)


# speedup vs baseline: 1.0169x; 1.0169x over previous
import functools

import jax
import jax.numpy as jnp
from jax import lax
from jax.experimental import pallas as pl
from jax.experimental.pallas import tpu as pltpu
from jax.experimental.pallas import tpu_sc as plsc

N = 10000
E = 320000
D = 128
H = 128
G = 16

NC = 2
NS = 16
NW = NC * NS
EPW = E // NW
EPT = E // NS
SLICES = ((0, 38400), (38400, 153600), (192000, 128000))
CHG = 200
CHS = 160
CHT = 160
NP = 10240
RPT = NP // NS

@functools.cache
def _sc_mesh():
    return plsc.VectorSubcoreMesh(
        core_axis_name="c", subcore_axis_name="s", num_cores=NC, num_subcores=NS)


def _prep_body(x_ref, ws_ref, wr_ref, be1_ref, p_ref, q_ref):
    xv = x_ref[...]
    p_ref[...] = jnp.dot(xv, ws_ref[...], preferred_element_type=jnp.float32)
    q_ref[...] = (jnp.dot(xv, wr_ref[...], preferred_element_type=jnp.float32)
                  + be1_ref[...])


def _prep(x, w1s, w1r, be1):
    bn = 2000
    return pl.pallas_call(
        _prep_body,
        grid=(N // bn,),
        in_specs=[
            pl.BlockSpec((bn, D), lambda i: (i, 0)),
            pl.BlockSpec((D, H), lambda i: (0, 0)),
            pl.BlockSpec((D, H), lambda i: (0, 0)),
            pl.BlockSpec((1, H), lambda i: (0, 0)),
        ],
        out_specs=[
            pl.BlockSpec((bn, H), lambda i: (i, 0)),
            pl.BlockSpec((bn, H), lambda i: (i, 0)),
        ],
        out_shape=[
            jax.ShapeDtypeStruct((N, H), jnp.float32),
            jax.ShapeDtypeStruct((N, H), jnp.float32),
        ],
    )(x, w1s, w1r, be1.reshape(1, H))


@functools.cache
def _sc_gather_kernel(ebase, esize):
    EPW_S = esize // NW
    NCHG = EPW_S // CHG
    assert NCHG * CHG == EPW_S and NCHG % 2 == 0

    @functools.partial(
        pl.kernel,
        out_type=jax.ShapeDtypeStruct((esize, H), jnp.float32),
        mesh=_sc_mesh(),
        scratch_types=[
            pltpu.VMEM((CHG,), jnp.int32),
            pltpu.VMEM((CHG,), jnp.int32),
            pltpu.VMEM((CHG,), jnp.int32),
            pltpu.VMEM((CHG,), jnp.int32),
            pltpu.VMEM((CHG, H), jnp.float32),
            pltpu.VMEM((CHG, H), jnp.float32),
            pltpu.VMEM((CHG, H), jnp.float32),
            pltpu.VMEM((CHG, H), jnp.float32),
            pltpu.SemaphoreType.DMA,
            pltpu.SemaphoreType.DMA,
            pltpu.SemaphoreType.DMA,
            pltpu.SemaphoreType.DMA,
        ],
    )
    def _sc_gather_impl(p_hbm, q_hbm, snd_hbm, rcv_hbm, s_hbm,
                        is0, is1, ir0, ir1, a0, a1, b0, b1,
                        gs0, gs1, ws0, ws1):
        IS, IR, A, B = (is0, is1), (ir0, ir1), (a0, a1), (b0, b1)
        GS, WS = (gs0, gs1), (ws0, ws1)
        wid = lax.axis_index("s") * NC + lax.axis_index("c")
        base = wid * EPW_S

        def load_idx(i, b):
            off = ebase + base + i * CHG
            pltpu.sync_copy(snd_hbm.at[pl.ds(off, CHG)], IS[b])
            pltpu.sync_copy(rcv_hbm.at[pl.ds(off, CHG)], IR[b])

        def start_gather(b):
            pltpu.async_copy(p_hbm.at[IS[b]], A[b], GS[b])
            pltpu.async_copy(q_hbm.at[IR[b]], B[b], GS[b])

        def wait_gather(b):
            pltpu.make_async_copy(p_hbm.at[IS[b]], A[b], GS[b]).wait()
            pltpu.make_async_copy(q_hbm.at[IR[b]], B[b], GS[b]).wait()

        def start_wb(i, b):
            off = base + i * CHG
            pltpu.async_copy(A[b], s_hbm.at[pl.ds(off, CHG)], WS[b])

        def wait_wb(b):
            pltpu.make_async_copy(A[b], s_hbm.at[pl.ds(base, CHG)], WS[b]).wait()

        def add(b):
            @plsc.parallel_loop(0, CHG, unroll=4)
            def _row(r):
                for j in range(H // 16):
                    sl = pl.ds(j * 16, 16)
                    A[b][r, sl] = A[b][r, sl] + B[b][r, sl]

        load_idx(0, 0)
        start_gather(0)
        load_idx(1, 1)
        start_gather(1)

        def pair(jj, carry):
            for b in (0, 1):
                wait_gather(b)
                add(b)
                start_wb(2 * jj + b, b)
            for b in (0, 1):
                nxt = 2 * jj + 2 + b

                @pl.when(nxt < NCHG)
                def _():
                    load_idx(nxt, b)
                    wait_wb(b)
                    start_gather(b)
            return carry

        lax.fori_loop(0, NCHG // 2, pair, 0)
        wait_wb(0)
        wait_wb(1)

    return _sc_gather_impl


def _sc_gather(p, q, senders, receivers, ebase, esize):
    return _sc_gather_kernel(ebase, esize)(p, q, senders, receivers)


def _edge_body(ea_ref, s_ref, w1_ref, w2_ref, b2_ref, w3_ref, b3_ref,
               msg_ref, eao_ref):
    ea = ea_ref[...]
    h = (jnp.dot(ea, w1_ref[...], preferred_element_type=jnp.float32)
         + s_ref[...])
    h = jnp.maximum(h, 0.0)
    h = jnp.dot(h, w2_ref[...], preferred_element_type=jnp.float32) + b2_ref[...]
    h = jnp.maximum(h, 0.0)
    m = jnp.dot(h, w3_ref[...], preferred_element_type=jnp.float32) + b3_ref[...]
    msg_ref[...] = m
    eao_ref[...] = ea + m


def _edge(edge_attr, s, w1e, We2, be2, We3, be3, ebase, esize):
    be = 1280
    bb = ebase // be
    return pl.pallas_call(
        _edge_body,
        grid=(esize // be,),
        in_specs=[
            pl.BlockSpec((be, D), lambda i: (i + bb, 0)),
            pl.BlockSpec((be, H), lambda i: (i, 0)),
            pl.BlockSpec((D, H), lambda i: (0, 0)),
            pl.BlockSpec((H, H), lambda i: (0, 0)),
            pl.BlockSpec((1, H), lambda i: (0, 0)),
            pl.BlockSpec((H, D), lambda i: (0, 0)),
            pl.BlockSpec((1, D), lambda i: (0, 0)),
        ],
        out_specs=[
            pl.BlockSpec((be, D), lambda i: (i, 0)),
            pl.BlockSpec((be, D), lambda i: (i, 0)),
        ],
        out_shape=[
            jax.ShapeDtypeStruct((esize, D), jnp.float32),
            jax.ShapeDtypeStruct((esize, D), jnp.float32),
        ],
    )(edge_attr, s, w1e, We2, be2.reshape(1, H), We3, be3.reshape(1, D))


@functools.cache
def _sc_scatter_kernel(ebase, esize):
    EPT_S = esize // NS
    NCHS = (EPT_S // CHS) // 2 * 2
    CHT_S = EPT_S - NCHS * CHS
    assert CHT_S in (0, 80, 160)

    @functools.partial(
        pl.kernel,
        out_type=jax.ShapeDtypeStruct((NC, NP, D), jnp.float32),
        mesh=_sc_mesh(),
        scratch_types=[
            pltpu.VMEM_SHARED((NP, D), jnp.float32),
            pltpu.VMEM((CHS,), jnp.int32),
            pltpu.VMEM((CHS,), jnp.int32),
            pltpu.VMEM((CHT,), jnp.int32),
            pltpu.VMEM((CHS, D), jnp.float32),
            pltpu.VMEM((CHS, D), jnp.float32),
            pltpu.SemaphoreType.DMA,
            pltpu.SemaphoreType.DMA,
            pltpu.SemaphoreType.DMA,
            pltpu.SemaphoreType.DMA,
        ],
    )
    def _sc_scatter_impl(msg_hbm, idx2_hbm, init_hbm, out_hbm,
                         acc, ix0, ix1, ixt, m0, m1, ls0, ls1, ss0, ss1):
        IX, M = (ix0, ix1), (m0, m1)
        LS, SS = (ls0, ls1), (ss0, ss1)
        cid = lax.axis_index("c")
        sid = lax.axis_index("s")

        pltpu.sync_copy(init_hbm.at[cid, pl.ds(sid * RPT, RPT)],
                        acc.at[pl.ds(sid * RPT, RPT)])
        plsc.subcore_barrier()

        base = sid * EPT_S

        def start_load(i, b):
            off = base + i * CHS
            pltpu.async_copy(idx2_hbm.at[pl.ds(cid * E + ebase + off, CHS)],
                             IX[b], LS[b])
            pltpu.async_copy(msg_hbm.at[pl.ds(off, CHS)], M[b], LS[b])

        def wait_load(b):
            pltpu.make_async_copy(idx2_hbm.at[pl.ds(base, CHS)], IX[b], LS[b]).wait()
            pltpu.make_async_copy(msg_hbm.at[pl.ds(base, CHS)], M[b], LS[b]).wait()

        def start_scatter(b):
            pltpu.async_copy(M[b], acc.at[IX[b]], SS[b], add=True)

        def wait_scatter(b):
            pltpu.make_async_copy(M[b], acc.at[IX[b]], SS[b]).wait()

        start_load(0, 0)
        start_load(1, 1)

        def pair(jj, carry):
            for b in (0, 1):
                wait_load(b)
                start_scatter(b)
            for b in (0, 1):
                nxt = 2 * jj + 2 + b

                @pl.when(nxt < NCHS)
                def _():
                    wait_scatter(b)
                    start_load(nxt, b)
            return carry

        lax.fori_loop(0, NCHS // 2, pair, 0)
        wait_scatter(0)
        wait_scatter(1)
        if CHT_S:
            toff = base + NCHS * CHS
            pltpu.sync_copy(idx2_hbm.at[pl.ds(cid * E + ebase + toff, CHT_S)],
                            ixt.at[pl.ds(0, CHT_S)])
            pltpu.sync_copy(msg_hbm.at[pl.ds(toff, CHT_S)],
                            m0.at[pl.ds(0, CHT_S)])
            pltpu.sync_copy(m0.at[pl.ds(0, CHT_S)], acc.at[ixt.at[pl.ds(0, CHT_S)]],
                            add=True)
        plsc.subcore_barrier()

        pltpu.sync_copy(acc.at[pl.ds(sid * RPT, RPT)],
                        out_hbm.at[cid, pl.ds(sid * RPT, RPT)])

    return _sc_scatter_impl


def _sc_scatter(messages, idx2, init, ebase, esize):
    return _sc_scatter_kernel(ebase, esize)(messages, idx2, init)


def _global_body(x_ref, b_ref, u_ref, g1x_ref, g1u_ref, bg1_ref,
                 g2_ref, bg2_ref, g3_ref, bg3_ref, mu_ref):
    xv = x_ref[...]
    b = b_ref[...]
    oh = (b == lax.broadcasted_iota(jnp.int32, (b.shape[0], D), 1)
          ).astype(jnp.float32)
    ub = jnp.dot(oh, u_ref[...], preferred_element_type=jnp.float32)
    h = (jnp.dot(xv, g1x_ref[...], preferred_element_type=jnp.float32)
         + jnp.dot(ub, g1u_ref[...], preferred_element_type=jnp.float32)
         + bg1_ref[...])
    h = jnp.maximum(h, 0.0)
    h = jnp.dot(h, g2_ref[...], preferred_element_type=jnp.float32) + bg2_ref[...]
    h = jnp.maximum(h, 0.0)
    mu_ref[...] = (jnp.dot(h, g3_ref[...], preferred_element_type=jnp.float32)
                   + bg3_ref[...])


def _global(x, batch2d, u_pad, g1x, g1u, bg1, Wg2, bg2, Wg3, bg3):
    bn = 2000
    full = lambda shape: pl.BlockSpec(shape, lambda i: tuple(0 for _ in shape))
    return pl.pallas_call(
        _global_body,
        grid=(N // bn,),
        in_specs=[
            pl.BlockSpec((bn, D), lambda i: (i, 0)),
            pl.BlockSpec((bn, 1), lambda i: (i, 0)),
            full((D, D)),
            full((D, H)), full((D, H)), full((1, H)),
            full((H, H)), full((1, H)),
            full((H, D)), full((1, D)),
        ],
        out_specs=pl.BlockSpec((bn, H), lambda i: (i, 0)),
        out_shape=jax.ShapeDtypeStruct((N, H), jnp.float32),
    )(x, batch2d, u_pad,
      g1x, g1u, bg1.reshape(1, H), Wg2, bg2.reshape(1, H),
      Wg3, bg3.reshape(1, D))


def _node_body(x_ref, rcv_ref, b_ref, mu_ref,
               n1x_ref, n1r_ref, n1u_ref, bn1_ref, n2_ref, bn2_ref,
               n3_ref, bn3_ref, out_ref):
    del b_ref
    xv = x_ref[...]
    rcv = rcv_ref[0] - rcv_ref[1]
    mu = mu_ref[...]
    h = (jnp.dot(xv, n1x_ref[...], preferred_element_type=jnp.float32)
         + jnp.dot(rcv, n1r_ref[...], preferred_element_type=jnp.float32)
         + jnp.dot(mu, n1u_ref[...], preferred_element_type=jnp.float32)
         + bn1_ref[...])
    h = jnp.maximum(h, 0.0)
    h = jnp.dot(h, n2_ref[...], preferred_element_type=jnp.float32) + bn2_ref[...]
    h = jnp.maximum(h, 0.0)
    gx = jnp.dot(h, n3_ref[...], preferred_element_type=jnp.float32) + bn3_ref[...]
    out_ref[...] = gx + xv


def _node(x, rcv, batch2d, mu,
          n1x, n1r, n1u, bn1, Wn2, bn2, Wn3, bn3):
    bn = 2000
    full = lambda shape: pl.BlockSpec(shape, lambda i: tuple(0 for _ in shape))
    return pl.pallas_call(
        _node_body,
        grid=(N // bn,),
        in_specs=[
            pl.BlockSpec((bn, D), lambda i: (i, 0)),
            pl.BlockSpec((NC, bn, D), lambda i: (0, i, 0)),
            pl.BlockSpec((bn, 1), lambda i: (i, 0)),
            pl.BlockSpec((bn, H), lambda i: (i, 0)),
            full((D, H)), full((D, H)), full((D, H)), full((1, H)),
            full((H, H)), full((1, H)),
            full((H, D)), full((1, D)),
        ],
        out_specs=pl.BlockSpec((bn, D), lambda i: (i, 0)),
        out_shape=jax.ShapeDtypeStruct((N, D), jnp.float32),
    )(x, rcv, batch2d, mu,
      n1x, n1r, n1u, bn1.reshape(1, H), Wn2, bn2.reshape(1, H),
      Wn3, bn3.reshape(1, D))


def kernel(x, senders, receivers, edge_attr, u, batch,
           We1, be1, We2, be2, We3, be3,
           Wg1, bg1, Wg2, bg2, Wg3, bg3,
           Wn1, bn1, Wn2, bn2, Wn3, bn3):
    w1e, w1s, w1r = We1[:D], We1[D:2 * D], We1[2 * D:]
    g1x, g1u = Wg1[:D], Wg1[D:]
    n1x, n1r, n1u = Wn1[:D], Wn1[D:2 * D], Wn1[2 * D:]

    p, q = _prep(x, w1s, w1r, be1)
    idx2 = jnp.concatenate([receivers, senders])

    ss = [_sc_gather(p, q, senders, receivers, b, z) for b, z in SLICES]
    eaos = []
    part = jnp.zeros((NC, NP, D), jnp.float32)
    for (b, z), s in zip(SLICES, ss):
        m, eao = _edge(edge_attr, s, w1e, We2, be2, We3, be3, b, z)
        part = _sc_scatter(m, idx2, part, b, z)
        eaos.append(eao)
    edge_attr_update = jnp.concatenate(eaos, axis=0)

    u_pad = jnp.zeros((D, D), jnp.float32).at[:G].set(u)
    batch2d = batch.reshape(N, 1)
    mu = _global(x, batch2d, u_pad, g1x, g1u, bg1, Wg2, bg2, Wg3, bg3)
    x_update = _node(x, part, batch2d, mu,
                     n1x, n1r, n1u, bn1, Wn2, bn2, Wn3, bn3)
    return (x_update, edge_attr_update)


# speedup vs baseline: 1.0180x; 1.0011x over previous
import functools

import jax
import jax.numpy as jnp
from jax import lax
from jax.experimental import pallas as pl
from jax.experimental.pallas import tpu as pltpu
from jax.experimental.pallas import tpu_sc as plsc

N = 10000
E = 320000
D = 128
H = 128
G = 16

NC = 2
NS = 16
NW = NC * NS
SLICES = ((0, 38400), (38400, 153600), (192000, 128000))
CHG = 200
CHS = 160
CHT = 160
NP = 10240
RPT = NP // NS

@functools.cache
def _sc_mesh():
    return plsc.VectorSubcoreMesh(
        core_axis_name="c", subcore_axis_name="s", num_cores=NC, num_subcores=NS)


def _prep_body(x_ref, ws_ref, wr_ref, be1_ref, p_ref, q_ref):
    xv = x_ref[...]
    p_ref[...] = jnp.dot(xv, ws_ref[...], preferred_element_type=jnp.float32)
    q_ref[...] = (jnp.dot(xv, wr_ref[...], preferred_element_type=jnp.float32)
                  + be1_ref[...])


def _prep(x, w1s, w1r, be1):
    bn = 2000
    return pl.pallas_call(
        _prep_body,
        grid=(N // bn,),
        in_specs=[
            pl.BlockSpec((bn, D), lambda i: (i, 0)),
            pl.BlockSpec((D, H), lambda i: (0, 0)),
            pl.BlockSpec((D, H), lambda i: (0, 0)),
            pl.BlockSpec((1, H), lambda i: (0, 0)),
        ],
        out_specs=[
            pl.BlockSpec((bn, H), lambda i: (i, 0)),
            pl.BlockSpec((bn, H), lambda i: (i, 0)),
        ],
        out_shape=[
            jax.ShapeDtypeStruct((N, H), jnp.float32),
            jax.ShapeDtypeStruct((N, H), jnp.float32),
        ],
    )(x, w1s, w1r, be1.reshape(1, H))


@functools.cache
def _sc_gather_kernel(ebase, esize):
    EPW_S = esize // NW
    NCHG = EPW_S // CHG
    assert NCHG * CHG == EPW_S and NCHG % 2 == 0

    @functools.partial(
        pl.kernel,
        out_type=jax.ShapeDtypeStruct((esize, H), jnp.float32),
        mesh=_sc_mesh(),
        scratch_types=[
            pltpu.VMEM((CHG,), jnp.int32),
            pltpu.VMEM((CHG,), jnp.int32),
            pltpu.VMEM((CHG,), jnp.int32),
            pltpu.VMEM((CHG,), jnp.int32),
            pltpu.VMEM((CHG, H), jnp.float32),
            pltpu.VMEM((CHG, H), jnp.float32),
            pltpu.VMEM((CHG, H), jnp.float32),
            pltpu.VMEM((CHG, H), jnp.float32),
            pltpu.SemaphoreType.DMA,
            pltpu.SemaphoreType.DMA,
            pltpu.SemaphoreType.DMA,
            pltpu.SemaphoreType.DMA,
        ],
    )
    def _sc_gather_impl(p_hbm, q_hbm, snd_hbm, rcv_hbm, s_hbm,
                        is0, is1, ir0, ir1, a0, a1, b0, b1,
                        gs0, gs1, ws0, ws1):
        IS, IR, A, B = (is0, is1), (ir0, ir1), (a0, a1), (b0, b1)
        GS, WS = (gs0, gs1), (ws0, ws1)
        wid = lax.axis_index("s") * NC + lax.axis_index("c")
        base = wid * EPW_S

        def load_idx(i, b):
            off = ebase + base + i * CHG
            pltpu.sync_copy(snd_hbm.at[pl.ds(off, CHG)], IS[b])
            pltpu.sync_copy(rcv_hbm.at[pl.ds(off, CHG)], IR[b])

        def start_gather(b):
            pltpu.async_copy(p_hbm.at[IS[b]], A[b], GS[b])
            pltpu.async_copy(q_hbm.at[IR[b]], B[b], GS[b])

        def wait_gather(b):
            pltpu.make_async_copy(p_hbm.at[IS[b]], A[b], GS[b]).wait()
            pltpu.make_async_copy(q_hbm.at[IR[b]], B[b], GS[b]).wait()

        def start_wb(i, b):
            off = base + i * CHG
            pltpu.async_copy(A[b], s_hbm.at[pl.ds(off, CHG)], WS[b])

        def wait_wb(b):
            pltpu.make_async_copy(A[b], s_hbm.at[pl.ds(base, CHG)], WS[b]).wait()

        def add(b):
            @plsc.parallel_loop(0, CHG, unroll=4)
            def _row(r):
                for j in range(H // 16):
                    sl = pl.ds(j * 16, 16)
                    A[b][r, sl] = A[b][r, sl] + B[b][r, sl]

        load_idx(0, 0)
        start_gather(0)
        load_idx(1, 1)
        start_gather(1)

        def pair(jj, carry):
            for b in (0, 1):
                wait_gather(b)
                add(b)
                start_wb(2 * jj + b, b)
            for b in (0, 1):
                nxt = 2 * jj + 2 + b

                @pl.when(nxt < NCHG)
                def _():
                    load_idx(nxt, b)
                    wait_wb(b)
                    start_gather(b)
            return carry

        lax.fori_loop(0, NCHG // 2, pair, 0)
        wait_wb(0)
        wait_wb(1)

    return _sc_gather_impl


def _sc_gather(p, q, senders, receivers, ebase, esize):
    return _sc_gather_kernel(ebase, esize)(p, q, senders, receivers)


def _edge_body(ea_ref, s_ref, w1_ref, w2_ref, b2_ref, w3_ref, b3_ref,
               msg_ref, eao_ref):
    ea = ea_ref[...]
    h = (jnp.dot(ea, w1_ref[...], preferred_element_type=jnp.float32)
         + s_ref[...])
    h = jnp.maximum(h, 0.0)
    h = jnp.dot(h, w2_ref[...], preferred_element_type=jnp.float32) + b2_ref[...]
    h = jnp.maximum(h, 0.0)
    m = jnp.dot(h, w3_ref[...], preferred_element_type=jnp.float32) + b3_ref[...]
    msg_ref[...] = m
    eao_ref[...] = ea + m


def _edge(edge_attr, s, w1e, We2, be2, We3, be3, ebase, esize):
    be = 1280
    bb = ebase // be
    return pl.pallas_call(
        _edge_body,
        grid=(esize // be,),
        in_specs=[
            pl.BlockSpec((be, D), lambda i: (i + bb, 0)),
            pl.BlockSpec((be, H), lambda i: (i, 0)),
            pl.BlockSpec((D, H), lambda i: (0, 0)),
            pl.BlockSpec((H, H), lambda i: (0, 0)),
            pl.BlockSpec((1, H), lambda i: (0, 0)),
            pl.BlockSpec((H, D), lambda i: (0, 0)),
            pl.BlockSpec((1, D), lambda i: (0, 0)),
        ],
        out_specs=[
            pl.BlockSpec((be, D), lambda i: (i, 0)),
            pl.BlockSpec((be, D), lambda i: (i, 0)),
        ],
        out_shape=[
            jax.ShapeDtypeStruct((esize, D), jnp.float32),
            jax.ShapeDtypeStruct((esize, D), jnp.float32),
        ],
    )(edge_attr, s, w1e, We2, be2.reshape(1, H), We3, be3.reshape(1, D))


@functools.cache
def _sc_scatter_kernel(ebase, esize):
    EPT_S = esize // NS
    NCHS = (EPT_S // CHS) // 2 * 2
    CHT_S = EPT_S - NCHS * CHS
    assert CHT_S in (0, 80, 160)

    @functools.partial(
        pl.kernel,
        out_type=jax.ShapeDtypeStruct((NC, NP, D), jnp.float32),
        mesh=_sc_mesh(),
        scratch_types=[
            pltpu.VMEM_SHARED((NP, D), jnp.float32),
            pltpu.VMEM((CHS,), jnp.int32),
            pltpu.VMEM((CHS,), jnp.int32),
            pltpu.VMEM((CHT,), jnp.int32),
            pltpu.VMEM((CHS, D), jnp.float32),
            pltpu.VMEM((CHS, D), jnp.float32),
            pltpu.SemaphoreType.DMA,
            pltpu.SemaphoreType.DMA,
            pltpu.SemaphoreType.DMA,
            pltpu.SemaphoreType.DMA,
        ],
    )
    def _sc_scatter_impl(msg_hbm, idx2_hbm, init_hbm, out_hbm,
                         acc, ix0, ix1, ixt, m0, m1, ls0, ls1, ss0, ss1):
        IX, M = (ix0, ix1), (m0, m1)
        LS, SS = (ls0, ls1), (ss0, ss1)
        cid = lax.axis_index("c")
        sid = lax.axis_index("s")

        pltpu.sync_copy(init_hbm.at[cid, pl.ds(sid * RPT, RPT)],
                        acc.at[pl.ds(sid * RPT, RPT)])
        plsc.subcore_barrier()

        base = sid * EPT_S

        def start_load(i, b):
            off = base + i * CHS
            pltpu.async_copy(idx2_hbm.at[pl.ds(cid * E + ebase + off, CHS)],
                             IX[b], LS[b])
            pltpu.async_copy(msg_hbm.at[pl.ds(off, CHS)], M[b], LS[b])

        def wait_load(b):
            pltpu.make_async_copy(idx2_hbm.at[pl.ds(base, CHS)], IX[b], LS[b]).wait()
            pltpu.make_async_copy(msg_hbm.at[pl.ds(base, CHS)], M[b], LS[b]).wait()

        def start_scatter(b):
            pltpu.async_copy(M[b], acc.at[IX[b]], SS[b], add=True)

        def wait_scatter(b):
            pltpu.make_async_copy(M[b], acc.at[IX[b]], SS[b]).wait()

        start_load(0, 0)
        start_load(1, 1)

        def pair(jj, carry):
            for b in (0, 1):
                wait_load(b)
                start_scatter(b)
            for b in (0, 1):
                nxt = 2 * jj + 2 + b

                @pl.when(nxt < NCHS)
                def _():
                    wait_scatter(b)
                    start_load(nxt, b)
            return carry

        lax.fori_loop(0, NCHS // 2, pair, 0)
        wait_scatter(0)
        wait_scatter(1)
        if CHT_S:
            toff = base + NCHS * CHS
            pltpu.sync_copy(idx2_hbm.at[pl.ds(cid * E + ebase + toff, CHT_S)],
                            ixt.at[pl.ds(0, CHT_S)])
            pltpu.sync_copy(msg_hbm.at[pl.ds(toff, CHT_S)],
                            m0.at[pl.ds(0, CHT_S)])
            pltpu.sync_copy(m0.at[pl.ds(0, CHT_S)], acc.at[ixt.at[pl.ds(0, CHT_S)]],
                            add=True)
        plsc.subcore_barrier()

        pltpu.sync_copy(acc.at[pl.ds(sid * RPT, RPT)],
                        out_hbm.at[cid, pl.ds(sid * RPT, RPT)])

    return _sc_scatter_impl


def _sc_scatter(messages, idx2, init, ebase, esize):
    return _sc_scatter_kernel(ebase, esize)(messages, idx2, init)


def _global_body(x_ref, b_ref, u_ref, g1x_ref, g1u_ref, bg1_ref,
                 g2_ref, bg2_ref, g3_ref, bg3_ref, mu_ref):
    xv = x_ref[...]
    b = b_ref[...]
    oh = (b == lax.broadcasted_iota(jnp.int32, (b.shape[0], D), 1)
          ).astype(jnp.float32)
    ub = jnp.dot(oh, u_ref[...], preferred_element_type=jnp.float32)
    h = (jnp.dot(xv, g1x_ref[...], preferred_element_type=jnp.float32)
         + jnp.dot(ub, g1u_ref[...], preferred_element_type=jnp.float32)
         + bg1_ref[...])
    h = jnp.maximum(h, 0.0)
    h = jnp.dot(h, g2_ref[...], preferred_element_type=jnp.float32) + bg2_ref[...]
    h = jnp.maximum(h, 0.0)
    mu_ref[...] = (jnp.dot(h, g3_ref[...], preferred_element_type=jnp.float32)
                   + bg3_ref[...])


def _global(x, batch2d, u_pad, g1x, g1u, bg1, Wg2, bg2, Wg3, bg3):
    bn = 2000
    full = lambda shape: pl.BlockSpec(shape, lambda i: tuple(0 for _ in shape))
    return pl.pallas_call(
        _global_body,
        grid=(N // bn,),
        in_specs=[
            pl.BlockSpec((bn, D), lambda i: (i, 0)),
            pl.BlockSpec((bn, 1), lambda i: (i, 0)),
            full((D, D)),
            full((D, H)), full((D, H)), full((1, H)),
            full((H, H)), full((1, H)),
            full((H, D)), full((1, D)),
        ],
        out_specs=pl.BlockSpec((bn, H), lambda i: (i, 0)),
        out_shape=jax.ShapeDtypeStruct((N, H), jnp.float32),
    )(x, batch2d, u_pad,
      g1x, g1u, bg1.reshape(1, H), Wg2, bg2.reshape(1, H),
      Wg3, bg3.reshape(1, D))


def _node_body(x_ref, rcv_ref, mu_ref,
               n1x_ref, n1r_ref, n1u_ref, bn1_ref, n2_ref, bn2_ref,
               n3_ref, bn3_ref, out_ref):
    xv = x_ref[...]
    rcv = rcv_ref[0] - rcv_ref[1]
    mu = mu_ref[...]
    h = (jnp.dot(xv, n1x_ref[...], preferred_element_type=jnp.float32)
         + jnp.dot(rcv, n1r_ref[...], preferred_element_type=jnp.float32)
         + jnp.dot(mu, n1u_ref[...], preferred_element_type=jnp.float32)
         + bn1_ref[...])
    h = jnp.maximum(h, 0.0)
    h = jnp.dot(h, n2_ref[...], preferred_element_type=jnp.float32) + bn2_ref[...]
    h = jnp.maximum(h, 0.0)
    gx = jnp.dot(h, n3_ref[...], preferred_element_type=jnp.float32) + bn3_ref[...]
    out_ref[...] = gx + xv


def _node(x, rcv, mu,
          n1x, n1r, n1u, bn1, Wn2, bn2, Wn3, bn3):
    bn = 2000
    full = lambda shape: pl.BlockSpec(shape, lambda i: tuple(0 for _ in shape))
    return pl.pallas_call(
        _node_body,
        grid=(N // bn,),
        in_specs=[
            pl.BlockSpec((bn, D), lambda i: (i, 0)),
            pl.BlockSpec((NC, bn, D), lambda i: (0, i, 0)),
            pl.BlockSpec((bn, H), lambda i: (i, 0)),
            full((D, H)), full((D, H)), full((D, H)), full((1, H)),
            full((H, H)), full((1, H)),
            full((H, D)), full((1, D)),
        ],
        out_specs=pl.BlockSpec((bn, D), lambda i: (i, 0)),
        out_shape=jax.ShapeDtypeStruct((N, D), jnp.float32),
    )(x, rcv, mu,
      n1x, n1r, n1u, bn1.reshape(1, H), Wn2, bn2.reshape(1, H),
      Wn3, bn3.reshape(1, D))


def kernel(x, senders, receivers, edge_attr, u, batch,
           We1, be1, We2, be2, We3, be3,
           Wg1, bg1, Wg2, bg2, Wg3, bg3,
           Wn1, bn1, Wn2, bn2, Wn3, bn3):
    w1e, w1s, w1r = We1[:D], We1[D:2 * D], We1[2 * D:]
    g1x, g1u = Wg1[:D], Wg1[D:]
    n1x, n1r, n1u = Wn1[:D], Wn1[D:2 * D], Wn1[2 * D:]

    p, q = _prep(x, w1s, w1r, be1)
    idx2 = jnp.concatenate([receivers, senders])

    ss = [_sc_gather(p, q, senders, receivers, b, z) for b, z in SLICES]
    eaos = []
    part = jnp.zeros((NC, NP, D), jnp.float32)
    for (b, z), s in zip(SLICES, ss):
        m, eao = _edge(edge_attr, s, w1e, We2, be2, We3, be3, b, z)
        part = _sc_scatter(m, idx2, part, b, z)
        eaos.append(eao)
    edge_attr_update = jnp.concatenate(eaos, axis=0)

    u_pad = jnp.zeros((D, D), jnp.float32).at[:G].set(u)
    batch2d = batch.reshape(N, 1)
    mu = _global(x, batch2d, u_pad, g1x, g1u, bg1, Wg2, bg2, Wg3, bg3)
    x_update = _node(x, part, mu,
                     n1x, n1r, n1u, bn1, Wn2, bn2, Wn3, bn3)
    return (x_update, edge_attr_update)
